# baseline (speedup 1.0000x reference)
import functools

import jax
import jax.numpy as jnp
from jax import lax
from jax.experimental import pallas as pl
from jax.experimental.pallas import tpu as pltpu
from jax.experimental.pallas import tpu_sc as plsc

K_STEPS = 4
N_NODES = 10000
D_FEAT = 128
N_EDGES = 320000

NUM_CORES = 2
NUM_SUBCORES = 16
HALF = D_FEAT // NUM_CORES
CHUNK = 128
N_CHUNKS = 160
E_PER_TILE = N_CHUNKS * CHUNK
E_PAD = E_PER_TILE * NUM_SUBCORES
N_PAD = 10240
ROWS_PER_TILE = N_PAD // NUM_SUBCORES
XCH = 80
CCH = 128
RING = 8
G_BUF = 4
S_BUF = 2
IDX_AHEAD = 6
ZR = 40
PF = plsc.PackFormat.INTERLEAVED


def _sc_diffusion(x, edges):
  mesh = plsc.VectorSubcoreMesh(core_axis_name="c", subcore_axis_name="s")
  out_t = [jax.ShapeDtypeStruct((N_NODES, D_FEAT), jnp.float32)
           for _ in range(K_STEPS)]

  @functools.partial(
      pl.kernel,
      out_type=out_t,
      mesh=mesh,
      compiler_params=pltpu.CompilerParams(use_tc_tiling_on_sc=False,
                                           needs_layout_passes=False),
      scratch_types=[
          pltpu.VMEM((RING, 3, CHUNK), jnp.int32),
          [pltpu.VMEM((CHUNK, HALF), jnp.bfloat16) for _ in range(G_BUF)],
          [pltpu.VMEM((CHUNK, HALF), jnp.float32) for _ in range(S_BUF)],
          pltpu.VMEM((ZR, HALF), jnp.float32),
          pltpu.VMEM_SHARED((N_PAD, HALF), jnp.float32),
          pltpu.VMEM_SHARED((N_PAD, HALF), jnp.bfloat16),
          [pltpu.SemaphoreType.DMA for _ in range(RING)],
          [pltpu.SemaphoreType.DMA for _ in range(G_BUF)],
          [pltpu.SemaphoreType.DMA for _ in range(S_BUF)],
      ],
  )
  def body(x_hbm, e_hbm, h1, h2, h3, h4,
           e_r, gbufs, sbufs, zero_v, acc, hbf,
           isems, gsems, ssems):
    c = lax.axis_index("c")
    s = lax.axis_index("s")
    row0 = s * ROWS_PER_TILE
    col0 = c * HALF
    last = NUM_SUBCORES - 1

    zvec = jnp.zeros((16,), jnp.float32)

    def zrow(r, _):
      for t in range(HALF // 16):
        zero_v[r, pl.ds(t * 16, 16)] = zvec
      return 0

    lax.fori_loop(0, ZR, zrow, 0)

    def zero_acc():
      for i in range(ROWS_PER_TILE // ZR):
        pltpu.sync_copy(zero_v, acc.at[pl.ds(row0 + i * ZR, ZR)])

    zero_acc()

    zvec_b = jnp.zeros((32,), jnp.bfloat16)

    def zrow_b(r, _):
      for t2 in range(HALF // 32):
        gbufs[0][r, pl.ds(t2 * 32, 32)] = zvec_b
      return 0

    lax.fori_loop(0, CHUNK, zrow_b, 0)
    for i in range(ROWS_PER_TILE // CHUNK):
      pltpu.sync_copy(gbufs[0], hbf.at[pl.ds(row0 + i * CHUNK, CHUNK)])

    def pack_rows(n_rows):
      def prow(r, _):
        for t2 in range(HALF // 32):
          a = sbufs[0][r, pl.ds(t2 * 32, 16)]
          b = sbufs[0][r, pl.ds(t2 * 32 + 16, 16)]
          gbufs[0][r, pl.ds(t2 * 32, 32)] = plsc.pack(a, b, format=PF)
        return 0

      lax.fori_loop(0, n_rows, prow, 0)

    def xload(i, _):
      pltpu.sync_copy(x_hbm.at[pl.ds(row0 + i * XCH, XCH), pl.ds(col0, HALF)],
                      sbufs[0].at[pl.ds(0, XCH)])
      pack_rows(XCH)
      pltpu.sync_copy(gbufs[0].at[pl.ds(0, XCH)],
                      hbf.at[pl.ds(row0 + i * XCH, XCH)])
      return 0

    n_x = jnp.where(s < last, ROWS_PER_TILE // XCH, 5)
    lax.fori_loop(0, n_x, xload, 0)
    plsc.subcore_barrier()

    def i_start(j, b):
      pltpu.async_copy(e_hbm.at[s, j], e_r.at[b], isems[b])

    def i_wait(j, b):
      pltpu.make_async_copy(e_hbm.at[s, j], e_r.at[b], isems[b]).wait()

    outs = [h1, h2, h3, h4]
    for k in range(K_STEPS):
      out = outs[k]

      def g_start(j, bg, bi):
        pltpu.async_copy(hbf.at[e_r.at[bi, 0]], gbufs[bg], gsems[bg])

      def g_wait(j, bg, bi):
        pltpu.make_async_copy(hbf.at[e_r.at[bi, 0]], gbufs[bg],
                              gsems[bg]).wait()

      def s_start(j, bs, bi):
        pltpu.async_copy(sbufs[bs], acc.at[e_r.at[bi, 1]], ssems[bs],
                         add=True)

      def s_wait(j, bs, bi):
        pltpu.make_async_copy(sbufs[bs], acc.at[e_r.at[bi, 1]],
                              ssems[bs]).wait()

      for j0 in range(IDX_AHEAD):
        i_start(j0, j0)
      for j0 in range(G_BUF):
        i_wait(j0, j0)
        g_start(j0, j0, j0)

      def round_body(r, _):
        for b in range(RING):
          j = r * RING + b
          bg = b % G_BUF
          bs = b % S_BUF

          @pl.when(j >= S_BUF)
          def _():
            s_wait(j - S_BUF, bs, (b - S_BUF) % RING)

          @pl.when(j + IDX_AHEAD < N_CHUNKS)
          def _():
            i_start(j + IDX_AHEAD, (b + IDX_AHEAD) % RING)

          g_wait(j, bg, b)

          def group_body(g, _):
            base = g * 16
            wv = plsc.bitcast(e_r[b, 2, pl.ds(base, 16)], jnp.float32)
            for e in range(16):
              wgt = wv[e]
              for t2 in range(HALF // 32):
                pk = gbufs[bg][base + e, pl.ds(t2 * 32, 32)]
                a, bb = plsc.unpack(pk, format=PF)
                sbufs[bs][base + e, pl.ds(t2 * 32, 16)] = a * wgt
                sbufs[bs][base + e, pl.ds(t2 * 32 + 16, 16)] = bb * wgt
            return 0

          lax.fori_loop(0, CHUNK // 16, group_body, 0)
          s_start(j, bs, b)

          @pl.when(j + G_BUF < N_CHUNKS)
          def _():
            bn = (b + G_BUF) % RING
            i_wait(j + G_BUF, bn)
            g_start(j + G_BUF, bg, bn)
        return 0

      lax.fori_loop(0, N_CHUNKS // RING, round_body, 0)
      for jt in range(N_CHUNKS - S_BUF, N_CHUNKS):
        s_wait(jt, jt % S_BUF, jt % RING)
      plsc.subcore_barrier()

      pltpu.sync_copy(acc.at[pl.ds(row0, 400)],
                      out.at[pl.ds(row0, 400), pl.ds(col0, HALF)])

      @pl.when(s < last)
      def _():
        pltpu.sync_copy(acc.at[pl.ds(row0 + 400, 240)],
                        out.at[pl.ds(row0 + 400, 240), pl.ds(col0, HALF)])

      if k < K_STEPS - 1:
        def repack(i, _):
          pltpu.sync_copy(acc.at[pl.ds(row0 + i * CCH, CCH)], sbufs[0])
          pack_rows(CCH)
          pltpu.sync_copy(gbufs[0], hbf.at[pl.ds(row0 + i * CCH, CCH)])
          return 0

        lax.fori_loop(0, ROWS_PER_TILE // CCH, repack, 0)
        zero_acc()
      plsc.subcore_barrier()

  return body(x, edges)


@jax.jit
def kernel(x, edge_index, edge_weight):
  src = edge_index[0].astype(jnp.int32)
  dst = edge_index[1].astype(jnp.int32)

  npad_e = E_PAD - N_EDGES
  src = jnp.concatenate([src, jnp.zeros((npad_e,), jnp.int32)])
  dst = jnp.concatenate([dst, jnp.full((npad_e,), N_NODES, jnp.int32)])
  wts = jnp.concatenate([edge_weight, jnp.zeros((npad_e,), jnp.float32)])

  wbits = jax.lax.bitcast_convert_type(wts, jnp.int32)
  edges = jnp.stack([
      src.reshape(NUM_SUBCORES, N_CHUNKS, CHUNK),
      dst.reshape(NUM_SUBCORES, N_CHUNKS, CHUNK),
      wbits.reshape(NUM_SUBCORES, N_CHUNKS, CHUNK),
  ], axis=2)

  hs = _sc_diffusion(x, edges)
  return (x,) + tuple(hs)


# speedup vs baseline: 1.8130x; 1.8130x over previous
import functools

import jax
import jax.numpy as jnp
from jax import lax
from jax.experimental import pallas as pl
from jax.experimental.pallas import tpu as pltpu
from jax.experimental.pallas import tpu_sc as plsc

K_STEPS = 4
N_NODES = 10000
D_FEAT = 128
N_EDGES = 320000

NUM_CORES = 2
NUM_SUBCORES = 16
HALF = D_FEAT // NUM_CORES
CHUNK = 128
N_CHUNKS = 160
E_PER_TILE = N_CHUNKS * CHUNK
E_PAD = E_PER_TILE * NUM_SUBCORES
N_PAD = 10240
ROWS_PER_TILE = N_PAD // NUM_SUBCORES
R_LO = 400
R_HI = 240
RING = 8
G_BUF = 2
S_BUF = 2
IDX_AHEAD = 6
ZR = 40


def _sc_diffusion(x, edges, w):
  mesh = plsc.VectorSubcoreMesh(core_axis_name="c", subcore_axis_name="s")
  out_t = [jax.ShapeDtypeStruct((N_NODES, D_FEAT), jnp.float32)
           for _ in range(K_STEPS)]

  @functools.partial(
      pl.kernel,
      out_type=out_t,
      mesh=mesh,
      compiler_params=pltpu.CompilerParams(use_tc_tiling_on_sc=False),
      scratch_types=[
          pltpu.VMEM((RING, 2, CHUNK), jnp.int32),
          pltpu.VMEM((RING, CHUNK), jnp.float32),
          [pltpu.VMEM((CHUNK, HALF), jnp.float32) for _ in range(G_BUF)],
          [pltpu.VMEM((CHUNK, HALF), jnp.float32) for _ in range(S_BUF)],
          pltpu.VMEM((ZR, HALF), jnp.float32),
          pltpu.VMEM_SHARED((N_PAD, HALF), jnp.float32),
          pltpu.VMEM_SHARED((N_PAD, HALF), jnp.float32),
          [pltpu.SemaphoreType.DMA for _ in range(RING)],
          [pltpu.SemaphoreType.DMA for _ in range(G_BUF)],
          [pltpu.SemaphoreType.DMA for _ in range(S_BUF)],
      ],
  )
  def body(x_hbm, e_hbm, w_hbm, h1, h2, h3, h4,
           e_r, w_r, gbufs, sbufs, zero_v, bufa, bufb,
           isems, gsems, ssems):
    c = lax.axis_index("c")
    s = lax.axis_index("s")
    row0 = s * ROWS_PER_TILE
    col0 = c * HALF

    zvec = jnp.zeros((16,), jnp.float32)

    def zrow(r, _):
      for t in range(HALF // 16):
        zero_v[r, pl.ds(t * 16, 16)] = zvec
      return 0

    lax.fori_loop(0, ZR, zrow, 0)

    def zero_buf(buf):
      for i in range(ROWS_PER_TILE // ZR):
        pltpu.sync_copy(zero_v, buf.at[pl.ds(row0 + i * ZR, ZR)])

    pltpu.sync_copy(x_hbm.at[pl.ds(row0, R_LO), pl.ds(col0, HALF)],
                    bufa.at[pl.ds(row0, R_LO)])

    @pl.when(s < NUM_SUBCORES - 1)
    def _():
      pltpu.sync_copy(x_hbm.at[pl.ds(row0 + R_LO, R_HI), pl.ds(col0, HALF)],
                      bufa.at[pl.ds(row0 + R_LO, R_HI)])

    @pl.when(s == NUM_SUBCORES - 1)
    def _():
      for i in range(R_HI // ZR):
        pltpu.sync_copy(zero_v, bufa.at[pl.ds(N_NODES + i * ZR, ZR)])

    zero_buf(bufb)
    plsc.subcore_barrier()

    def i_start(j, b):
      pltpu.async_copy(e_hbm.at[s, j], e_r.at[b], isems[b])
      pltpu.async_copy(w_hbm.at[s, j], w_r.at[b], isems[b])

    def i_wait(j, b):
      pltpu.make_async_copy(e_hbm.at[s, j], e_r.at[b], isems[b]).wait()
      pltpu.make_async_copy(w_hbm.at[s, j], w_r.at[b], isems[b]).wait()

    bufs = [bufa, bufb, bufa, bufb, bufa]
    outs = [h1, h2, h3, h4]
    for k in range(K_STEPS):
      prev = bufs[k]
      nxt = bufs[k + 1]
      out = outs[k]

      def g_start(j, bg, bi):
        pltpu.async_copy(prev.at[e_r.at[bi, 0]], gbufs[bg], gsems[bg])

      def g_wait(j, bg, bi):
        pltpu.make_async_copy(prev.at[e_r.at[bi, 0]], gbufs[bg],
                              gsems[bg]).wait()

      def s_start(j, bs, bi):
        pltpu.async_copy(sbufs[bs], nxt.at[e_r.at[bi, 1]], ssems[bs],
                         add=True)

      def s_wait(j, bs, bi):
        pltpu.make_async_copy(sbufs[bs], nxt.at[e_r.at[bi, 1]],
                              ssems[bs]).wait()

      for j0 in range(IDX_AHEAD):
        i_start(j0, j0)
      for j0 in range(G_BUF):
        i_wait(j0, j0)
        g_start(j0, j0, j0)

      def round_body(r, _):
        for b in range(RING):
          j = r * RING + b
          bg = b % G_BUF
          bs = b % S_BUF

          @pl.when(j >= S_BUF)
          def _():
            s_wait(j - S_BUF, bs, (b - S_BUF) % RING)

          @pl.when(j + IDX_AHEAD < N_CHUNKS)
          def _():
            i_start(j + IDX_AHEAD, (b + IDX_AHEAD) % RING)

          g_wait(j, bg, b)

          def group_body(g, _):
            base = g * 16
            wv = w_r[b, pl.ds(base, 16)]
            for e in range(16):
              wgt = wv[e]
              for t in range(HALF // 16):
                sl = pl.ds(t * 16, 16)
                sbufs[bs][base + e, sl] = gbufs[bg][base + e, sl] * wgt
            return 0

          lax.fori_loop(0, CHUNK // 16, group_body, 0)
          s_start(j, bs, b)

          @pl.when(j + G_BUF < N_CHUNKS)
          def _():
            bn = (b + G_BUF) % RING
            i_wait(j + G_BUF, bn)
            g_start(j + G_BUF, bg, bn)
        return 0

      lax.fori_loop(0, N_CHUNKS // RING, round_body, 0)
      for jt in range(N_CHUNKS - S_BUF, N_CHUNKS):
        s_wait(jt, jt % S_BUF, jt % RING)
      plsc.subcore_barrier()

      pltpu.sync_copy(nxt.at[pl.ds(row0, R_LO)],
                      out.at[pl.ds(row0, R_LO), pl.ds(col0, HALF)])

      @pl.when(s < NUM_SUBCORES - 1)
      def _():
        pltpu.sync_copy(nxt.at[pl.ds(row0 + R_LO, R_HI)],
                        out.at[pl.ds(row0 + R_LO, R_HI), pl.ds(col0, HALF)])

      zero_buf(prev)
      plsc.subcore_barrier()

  return body(x, edges, w)


@jax.jit
def kernel(x, edge_index, edge_weight):
  src = edge_index[0].astype(jnp.int32)
  dst = edge_index[1].astype(jnp.int32)

  npad_e = E_PAD - N_EDGES
  src = jnp.concatenate([src, jnp.zeros((npad_e,), jnp.int32)])
  dst = jnp.concatenate([dst, jnp.full((npad_e,), N_NODES, jnp.int32)])
  wts = jnp.concatenate([edge_weight, jnp.zeros((npad_e,), jnp.float32)])

  edges = jnp.stack([
      src.reshape(NUM_SUBCORES, N_CHUNKS, CHUNK),
      dst.reshape(NUM_SUBCORES, N_CHUNKS, CHUNK),
  ], axis=2)
  w = wts.reshape(NUM_SUBCORES, N_CHUNKS, CHUNK)

  hs = _sc_diffusion(x, edges, w)
  return (x,) + tuple(hs)


# speedup vs baseline: 1.8203x; 1.0040x over previous
import functools

import jax
import jax.numpy as jnp
from jax import lax
from jax.experimental import pallas as pl
from jax.experimental.pallas import tpu as pltpu
from jax.experimental.pallas import tpu_sc as plsc

K_STEPS = 4
N_NODES = 10000
D_FEAT = 128
N_EDGES = 320000

NUM_CORES = 2
NUM_SUBCORES = 16
HALF = D_FEAT // NUM_CORES
CHUNK = 128
N_CHUNKS = 160
E_PER_TILE = N_CHUNKS * CHUNK
E_PAD = E_PER_TILE * NUM_SUBCORES
N_PAD = 10240
ROWS_PER_TILE = N_PAD // NUM_SUBCORES
R_LO = 400
R_HI = 240
RING = 8
G_BUF = 2
S_BUF = 2
IDX_AHEAD = 6
ZR = 40


def _sc_diffusion(x_split, src_idx, dst_idx, w):
  mesh = plsc.VectorSubcoreMesh(core_axis_name="c", subcore_axis_name="s")
  out_t = [jax.ShapeDtypeStruct((N_NODES, D_FEAT), jnp.float32)
           for _ in range(K_STEPS)]

  @functools.partial(
      pl.kernel,
      out_type=out_t,
      mesh=mesh,
      compiler_params=pltpu.CompilerParams(use_tc_tiling_on_sc=False),
      scratch_types=[
          pltpu.VMEM((RING, CHUNK), jnp.int32),
          pltpu.VMEM((RING, CHUNK), jnp.int32),
          pltpu.VMEM((RING, CHUNK), jnp.float32),
          [pltpu.VMEM((CHUNK, HALF), jnp.float32) for _ in range(G_BUF)],
          [pltpu.VMEM((CHUNK, HALF), jnp.float32) for _ in range(S_BUF)],
          pltpu.VMEM((ZR, HALF), jnp.float32),
          pltpu.VMEM_SHARED((N_PAD, HALF), jnp.float32),
          pltpu.VMEM_SHARED((N_PAD, HALF), jnp.float32),
          [pltpu.SemaphoreType.DMA for _ in range(RING)],
          [pltpu.SemaphoreType.DMA for _ in range(G_BUF)],
          [pltpu.SemaphoreType.DMA for _ in range(S_BUF)],
      ],
  )
  def body(x_hbm, src_hbm, dst_hbm, w_hbm, h1, h2, h3, h4,
           src_r, dst_r, w_r, gbufs, sbufs, zero_v, bufa, bufb,
           isems, gsems, ssems):
    c = lax.axis_index("c")
    s = lax.axis_index("s")
    row0 = s * ROWS_PER_TILE
    col0 = c * HALF

    zvec = jnp.zeros((16,), jnp.float32)

    def zrow(r, _):
      for t in range(HALF // 16):
        zero_v[r, pl.ds(t * 16, 16)] = zvec
      return 0

    lax.fori_loop(0, ZR, zrow, 0)

    def zero_buf(buf):
      for i in range(ROWS_PER_TILE // ZR):
        pltpu.sync_copy(zero_v, buf.at[pl.ds(row0 + i * ZR, ZR)])

    pltpu.sync_copy(x_hbm.at[pl.ds(row0, R_LO), pl.ds(col0, HALF)],
                    bufa.at[pl.ds(row0, R_LO)])

    @pl.when(s < NUM_SUBCORES - 1)
    def _():
      pltpu.sync_copy(x_hbm.at[pl.ds(row0 + R_LO, R_HI), pl.ds(col0, HALF)],
                      bufa.at[pl.ds(row0 + R_LO, R_HI)])

    @pl.when(s == NUM_SUBCORES - 1)
    def _():
      for i in range(R_HI // ZR):
        pltpu.sync_copy(zero_v, bufa.at[pl.ds(N_NODES + i * ZR, ZR)])

    zero_buf(bufb)
    plsc.subcore_barrier()

    def i_start(j, b):
      pltpu.async_copy(src_hbm.at[s, j], src_r.at[b], isems[b])
      pltpu.async_copy(dst_hbm.at[s, j], dst_r.at[b], isems[b])
      pltpu.async_copy(w_hbm.at[s, j], w_r.at[b], isems[b])

    def i_wait(j, b):
      pltpu.make_async_copy(src_hbm.at[s, j], src_r.at[b], isems[b]).wait()
      pltpu.make_async_copy(dst_hbm.at[s, j], dst_r.at[b], isems[b]).wait()
      pltpu.make_async_copy(w_hbm.at[s, j], w_r.at[b], isems[b]).wait()

    bufs = [bufa, bufb, bufa, bufb, bufa]
    outs = [h1, h2, h3, h4]
    for k in range(K_STEPS):
      prev = bufs[k]
      nxt = bufs[k + 1]
      out = outs[k]

      def g_start(j, bg, bi):
        pltpu.async_copy(prev.at[src_r.at[bi]], gbufs[bg], gsems[bg])

      def g_wait(j, bg, bi):
        pltpu.make_async_copy(prev.at[src_r.at[bi]], gbufs[bg],
                              gsems[bg]).wait()

      def s_start(j, bs, bi):
        pltpu.async_copy(sbufs[bs], nxt.at[dst_r.at[bi]], ssems[bs],
                         add=True)

      def s_wait(j, bs, bi):
        pltpu.make_async_copy(sbufs[bs], nxt.at[dst_r.at[bi]],
                              ssems[bs]).wait()

      for j0 in range(IDX_AHEAD):
        i_start(j0, j0)
      for j0 in range(G_BUF):
        i_wait(j0, j0)
        g_start(j0, j0, j0)

      def round_body(r, _):
        for b in range(RING):
          j = r * RING + b
          bg = b % G_BUF
          bs = b % S_BUF

          @pl.when(j >= S_BUF)
          def _():
            s_wait(j - S_BUF, bs, (b - S_BUF) % RING)

          @pl.when(j + IDX_AHEAD < N_CHUNKS)
          def _():
            i_start(j + IDX_AHEAD, (b + IDX_AHEAD) % RING)

          g_wait(j, bg, b)

          def group_body(g, _):
            base = g * 16
            wv = w_r[b, pl.ds(base, 16)]
            for e in range(16):
              wgt = wv[e]
              for t in range(HALF // 16):
                sl = pl.ds(t * 16, 16)
                sbufs[bs][base + e, sl] = gbufs[bg][base + e, sl] * wgt
            return 0

          lax.fori_loop(0, CHUNK // 16, group_body, 0)
          s_start(j, bs, b)

          @pl.when(j + G_BUF < N_CHUNKS)
          def _():
            bn = (b + G_BUF) % RING
            i_wait(j + G_BUF, bn)
            g_start(j + G_BUF, bg, bn)
        return 0

      lax.fori_loop(0, N_CHUNKS // RING, round_body, 0)
      for jt in range(N_CHUNKS - S_BUF, N_CHUNKS):
        s_wait(jt, jt % S_BUF, jt % RING)
      plsc.subcore_barrier()

      pltpu.sync_copy(nxt.at[pl.ds(row0, R_LO)],
                      out.at[pl.ds(row0, R_LO), pl.ds(col0, HALF)])

      @pl.when(s < NUM_SUBCORES - 1)
      def _():
        pltpu.sync_copy(nxt.at[pl.ds(row0 + R_LO, R_HI)],
                        out.at[pl.ds(row0 + R_LO, R_HI), pl.ds(col0, HALF)])

      zero_buf(prev)
      plsc.subcore_barrier()

  return body(x_split, src_idx, dst_idx, w)


@jax.jit
def kernel(x, edge_index, edge_weight):
  src = edge_index[0].astype(jnp.int32)
  dst = edge_index[1].astype(jnp.int32)

  npad_e = E_PAD - N_EDGES
  src = jnp.concatenate([src, jnp.zeros((npad_e,), jnp.int32)])
  dst = jnp.concatenate([dst, jnp.full((npad_e,), N_NODES, jnp.int32)])
  wts = jnp.concatenate([edge_weight, jnp.zeros((npad_e,), jnp.float32)])

  src_idx = src.reshape(NUM_SUBCORES, N_CHUNKS, CHUNK)
  dst_idx = dst.reshape(NUM_SUBCORES, N_CHUNKS, CHUNK)
  w = wts.reshape(NUM_SUBCORES, N_CHUNKS, CHUNK)

  hs = _sc_diffusion(x, src_idx, dst_idx, w)
  return (x,) + tuple(hs)
